# baseline (speedup 1.0000x reference)
import jax
import jax.numpy as jnp
from jax import lax
from jax.experimental import pallas as pl
from jax.experimental.pallas import tpu as pltpu
from jax.experimental.pallas import tpu_sc as plsc

N_NODES = 10000
N_EDGES = 320000
D_FEAT = 128
H = D_FEAT // 2
NC = 2
NS = 16
EPT = N_EDGES // NS
CHUNK = 128
NCHUNK = 158
HCHUNK = NCHUNK // 2
EPT_PAD = NCHUNK * CHUNK
N_PAD = 10008
TRASH = N_NODES
RPT = 624
TAIL = N_NODES - NS * RPT
IB = 104
NB = 3


def _gin_body(feat, srcp, dstp, eps16, out, hbmtab, table, acc, src_v, dst_v,
              rows, eps_v, sem_g, sem_s, sem_i, sem_t):
    c = lax.axis_index("c")
    s = lax.axis_index("s")
    hbm_half = hbmtab.at[pl.ds(c * N_NODES, N_NODES)]

    tbl = pltpu.async_copy(feat.at[pl.ds(s * RPT, RPT), pl.ds(c * H, H)],
                           table.at[pl.ds(s * RPT, RPT)], sem_t)
    idx_src = pltpu.async_copy(srcp.at[s, pl.ds(0, HCHUNK)], src_v, sem_i)
    idx_dst = pltpu.async_copy(dstp.at[s, pl.ds(0, HCHUNK)], dst_v, sem_i)

    pltpu.sync_copy(eps16, eps_v)
    scale = eps_v[...] + 1.0

    @pl.when(s == NS - 1)
    def _():
        pltpu.sync_copy(feat.at[pl.ds(NS * RPT, TAIL), pl.ds(c * H, H)],
                        table.at[pl.ds(NS * RPT, TAIL)])

    tbl.wait()

    pltpu.sync_copy(table.at[pl.ds(s * RPT, RPT)],
                    hbm_half.at[pl.ds(s * RPT, RPT)])

    @pl.when(s == NS - 1)
    def _():
        pltpu.sync_copy(table.at[pl.ds(NS * RPT, TAIL)],
                        hbm_half.at[pl.ds(NS * RPT, TAIL)])

    def init_range(r0, nrows):
        pltpu.sync_copy(table.at[pl.ds(r0, nrows)],
                        rows.at[NB - 1, pl.ds(0, nrows)])

        def row_scale(r, carry):
            for j in range(H // 16):
                rows[NB - 1, r, pl.ds(j * 16, 16)] = (
                    rows[NB - 1, r, pl.ds(j * 16, 16)] * scale)
            return carry

        lax.fori_loop(0, nrows, row_scale, 0)
        pltpu.sync_copy(rows.at[NB - 1, pl.ds(0, nrows)],
                        acc.at[pl.ds(r0, nrows)])

    for p in range(RPT // IB):
        init_range(s * RPT + p * IB, IB)

    @pl.when(s == NS - 1)
    def _():
        init_range(NS * RPT, TAIL)

    plsc.subcore_barrier()
    idx_src.wait()
    idx_dst.wait()

    def start_gather(j, buf):
        @pl.when(lax.rem(j, 2) == 0)
        def _():
            pltpu.async_copy(table.at[src_v.at[j]], rows.at[buf],
                             sem_g.at[buf])

        @pl.when(lax.rem(j, 2) == 1)
        def _():
            pltpu.async_copy(hbm_half.at[src_v.at[j]], rows.at[buf],
                             sem_g.at[buf])

    def wait_gather(j, buf):
        @pl.when(lax.rem(j, 2) == 0)
        def _():
            pltpu.make_async_copy(table.at[src_v.at[j]], rows.at[buf],
                                  sem_g.at[buf]).wait()

        @pl.when(lax.rem(j, 2) == 1)
        def _():
            pltpu.make_async_copy(hbm_half.at[src_v.at[j]], rows.at[buf],
                                  sem_g.at[buf]).wait()

    def run_half():
        start_gather(0, 0)

        def chunk_body(k, carry):
            b = lax.rem(k, NB)
            fb = lax.rem(k + 1, NB)

            @pl.when(k >= 2)
            def _():
                pltpu.make_async_copy(rows.at[fb], acc.at[dst_v.at[k - 2]],
                                      sem_s.at[fb]).wait()

            @pl.when(k + 1 < HCHUNK)
            def _():
                start_gather(k + 1, fb)

            wait_gather(k, b)
            pltpu.async_copy(rows.at[b], acc.at[dst_v.at[k]], sem_s.at[b],
                             add=True)
            return carry

        lax.fori_loop(0, HCHUNK, chunk_body, 0)
        for j in (HCHUNK - 2, HCHUNK - 1):
            pltpu.make_async_copy(rows.at[j % NB], acc.at[dst_v.at[j]],
                                  sem_s.at[j % NB]).wait()

    run_half()
    pltpu.sync_copy(srcp.at[s, pl.ds(HCHUNK, HCHUNK)], src_v)
    pltpu.sync_copy(dstp.at[s, pl.ds(HCHUNK, HCHUNK)], dst_v)
    run_half()
    plsc.subcore_barrier()

    pltpu.sync_copy(acc.at[pl.ds(s * RPT, RPT)],
                    out.at[pl.ds(s * RPT, RPT), pl.ds(c * H, H)])

    @pl.when(s == NS - 1)
    def _():
        pltpu.sync_copy(acc.at[pl.ds(NS * RPT, TAIL)],
                        out.at[pl.ds(NS * RPT, TAIL), pl.ds(c * H, H)])


@jax.jit
def kernel(feat, edge_index, eps):
    src = edge_index[0]
    dst = edge_index[1]

    pad = EPT_PAD - EPT
    srcp = jnp.pad(src.reshape(NS, EPT), ((0, 0), (0, pad)))
    srcp = srcp.reshape(NS, NCHUNK, CHUNK)
    dstp = jnp.pad(dst.reshape(NS, EPT), ((0, 0), (0, pad)),
                   constant_values=TRASH).reshape(NS, NCHUNK, CHUNK)

    eps16 = jnp.broadcast_to(eps, (16,))

    mesh = plsc.VectorSubcoreMesh(core_axis_name="c", subcore_axis_name="s")
    out = pl.kernel(
        _gin_body,
        out_type=(jax.ShapeDtypeStruct((N_NODES, D_FEAT), jnp.float32),
                  jax.ShapeDtypeStruct((NC * N_NODES, H), jnp.float32)),
        mesh=mesh,
        compiler_params=pltpu.CompilerParams(use_tc_tiling_on_sc=False),
        scratch_types=[
            pltpu.VMEM_SHARED((N_NODES, H), jnp.float32),
            pltpu.VMEM_SHARED((N_PAD, H), jnp.float32),
            pltpu.VMEM((HCHUNK, CHUNK), jnp.int32),
            pltpu.VMEM((HCHUNK, CHUNK), jnp.int32),
            pltpu.VMEM((NB, CHUNK, H), jnp.float32),
            pltpu.VMEM((16,), jnp.float32),
            pltpu.SemaphoreType.DMA((NB,)),
            pltpu.SemaphoreType.DMA((NB,)),
            pltpu.SemaphoreType.DMA,
            pltpu.SemaphoreType.DMA,
        ],
    )(feat, srcp, dstp, eps16)
    return out[0]


# speedup vs baseline: 1.2823x; 1.2823x over previous
import jax
import jax.numpy as jnp
from jax import lax
from jax.experimental import pallas as pl
from jax.experimental.pallas import tpu as pltpu
from jax.experimental.pallas import tpu_sc as plsc

N_NODES = 10000
N_EDGES = 320000
D_FEAT = 128
H = D_FEAT // 2
NC = 2
NS = 16
EPT = N_EDGES // NS
CHUNK = 128
NCHUNK = 158
HCHUNK = NCHUNK // 2
EPT_PAD = NCHUNK * CHUNK
N_PAD = 10008
TRASH = N_NODES
RPT = 624
TAIL = N_NODES - NS * RPT
IB = 104
NB = 3


def _gin_body(feat, srcp, dstp, eps16, out, table, acc, src_v, dst_v, rows,
              eps_v, sem_g, sem_s, sem_i, sem_t):
    c = lax.axis_index("c")
    s = lax.axis_index("s")

    tbl = pltpu.async_copy(feat.at[pl.ds(s * RPT, RPT), pl.ds(c * H, H)],
                           table.at[pl.ds(s * RPT, RPT)], sem_t)
    idx_src = pltpu.async_copy(srcp.at[s, pl.ds(0, HCHUNK)], src_v, sem_i)
    idx_dst = pltpu.async_copy(dstp.at[s, pl.ds(0, HCHUNK)], dst_v, sem_i)

    pltpu.sync_copy(eps16, eps_v)
    scale = eps_v[...] + 1.0

    @pl.when(s == NS - 1)
    def _():
        pltpu.sync_copy(feat.at[pl.ds(NS * RPT, TAIL), pl.ds(c * H, H)],
                        table.at[pl.ds(NS * RPT, TAIL)])

    tbl.wait()

    def init_range(r0, nrows):
        pltpu.sync_copy(table.at[pl.ds(r0, nrows)],
                        rows.at[NB - 1, pl.ds(0, nrows)])

        def row_scale(r, carry):
            for j in range(H // 16):
                rows[NB - 1, r, pl.ds(j * 16, 16)] = (
                    rows[NB - 1, r, pl.ds(j * 16, 16)] * scale)
            return carry

        lax.fori_loop(0, nrows, row_scale, 0)
        pltpu.sync_copy(rows.at[NB - 1, pl.ds(0, nrows)],
                        acc.at[pl.ds(r0, nrows)])

    for p in range(RPT // IB):
        init_range(s * RPT + p * IB, IB)

    @pl.when(s == NS - 1)
    def _():
        init_range(NS * RPT, TAIL)

    plsc.subcore_barrier()
    idx_src.wait()
    idx_dst.wait()

    def run_half():
        pltpu.async_copy(table.at[src_v.at[0]], rows.at[0], sem_g.at[0])

        def chunk_body(k, carry):
            b = lax.rem(k, NB)
            fb = lax.rem(k + 1, NB)

            @pl.when(k >= 2)
            def _():
                pltpu.make_async_copy(rows.at[fb], acc.at[dst_v.at[k - 2]],
                                      sem_s.at[fb]).wait()

            @pl.when(k + 1 < HCHUNK)
            def _():
                pltpu.async_copy(table.at[src_v.at[k + 1]], rows.at[fb],
                                 sem_g.at[fb])

            pltpu.make_async_copy(table.at[src_v.at[k]], rows.at[b],
                                  sem_g.at[b]).wait()
            pltpu.async_copy(rows.at[b], acc.at[dst_v.at[k]], sem_s.at[b],
                             add=True)
            return carry

        lax.fori_loop(0, HCHUNK, chunk_body, 0)
        for j in (HCHUNK - 2, HCHUNK - 1):
            pltpu.make_async_copy(rows.at[j % NB], acc.at[dst_v.at[j]],
                                  sem_s.at[j % NB]).wait()

    run_half()
    pltpu.sync_copy(srcp.at[s, pl.ds(HCHUNK, HCHUNK)], src_v)
    pltpu.sync_copy(dstp.at[s, pl.ds(HCHUNK, HCHUNK)], dst_v)
    run_half()
    plsc.subcore_barrier()

    pltpu.sync_copy(acc.at[pl.ds(s * RPT, RPT)],
                    out.at[pl.ds(s * RPT, RPT), pl.ds(c * H, H)])

    @pl.when(s == NS - 1)
    def _():
        pltpu.sync_copy(acc.at[pl.ds(NS * RPT, TAIL)],
                        out.at[pl.ds(NS * RPT, TAIL), pl.ds(c * H, H)])


@jax.jit
def kernel(feat, edge_index, eps):
    src = edge_index[0]
    dst = edge_index[1]

    pad = EPT_PAD - EPT
    srcp = jnp.pad(src.reshape(NS, EPT), ((0, 0), (0, pad)))
    srcp = srcp.reshape(NS, NCHUNK, CHUNK)
    dstp = jnp.pad(dst.reshape(NS, EPT), ((0, 0), (0, pad)),
                   constant_values=TRASH).reshape(NS, NCHUNK, CHUNK)

    eps16 = jnp.broadcast_to(eps, (16,))

    mesh = plsc.VectorSubcoreMesh(core_axis_name="c", subcore_axis_name="s")
    out = pl.kernel(
        _gin_body,
        out_type=jax.ShapeDtypeStruct((N_NODES, D_FEAT), jnp.float32),
        mesh=mesh,
        compiler_params=pltpu.CompilerParams(use_tc_tiling_on_sc=False),
        scratch_types=[
            pltpu.VMEM_SHARED((N_NODES, H), jnp.float32),
            pltpu.VMEM_SHARED((N_PAD, H), jnp.float32),
            pltpu.VMEM((HCHUNK, CHUNK), jnp.int32),
            pltpu.VMEM((HCHUNK, CHUNK), jnp.int32),
            pltpu.VMEM((NB, CHUNK, H), jnp.float32),
            pltpu.VMEM((16,), jnp.float32),
            pltpu.SemaphoreType.DMA((NB,)),
            pltpu.SemaphoreType.DMA((NB,)),
            pltpu.SemaphoreType.DMA,
            pltpu.SemaphoreType.DMA,
        ],
    )(feat, srcp, dstp, eps16)
    return out
